# Initial kernel scaffold; baseline (speedup 1.0000x reference)
import jax
import jax.numpy as jnp
from jax.experimental import pallas as pl


def kernel(x, edge_index, W1, b1, g1, be1, W2, b2, g2, be2, W3, b3):
    raise NotImplementedError("write your pallas kernel here")


# speedup vs baseline: 14.7581x; 14.7581x over previous
import functools

import jax
import jax.numpy as jnp
from jax import lax
from jax.experimental import pallas as pl
from jax.experimental.pallas import tpu as pltpu
from jax.experimental.pallas import tpu_sc as plsc

N = 10000
EPS = 1e-5
NC = 2
NS = 16
NW = NC * NS
G = 128
NP = 10240
RPT = NP // NS
R = 400
GRID = N // R

_mesh = lambda: plsc.VectorSubcoreMesh(
    core_axis_name="c", subcore_axis_name="s", num_cores=NC, num_subcores=NS)


@functools.lru_cache(maxsize=None)
def _sc_deg(e_pad):
    epw = e_pad // NW
    ch = epw // G

    @functools.partial(
        pl.kernel,
        out_type=jax.ShapeDtypeStruct((NC * NP,), jnp.float32),
        mesh=_mesh(),
        scratch_types=[
            pltpu.VMEM((G,), jnp.int32),
            pltpu.VMEM((G,), jnp.float32),
            pltpu.VMEM((RPT,), jnp.float32),
            pltpu.VMEM_SHARED((NP,), jnp.float32),
        ],
    )
    def k(dst_hbm, out_hbm, idx_v, upd_v, buf_v, acc_sh):
        c = lax.axis_index("c")
        s = lax.axis_index("s")
        wid = s * NC + c
        ones16 = jnp.ones((16,), jnp.float32)
        zeros16 = jnp.zeros((16,), jnp.float32)

        def fill_upd(i, carry):
            upd_v[pl.ds(16 * i, 16)] = ones16
            return carry

        lax.fori_loop(0, G // 16, fill_upd, 0)

        def fill_buf(i, carry):
            buf_v[pl.ds(16 * i, 16)] = zeros16
            return carry

        lax.fori_loop(0, RPT // 16, fill_buf, 0)
        pltpu.sync_copy(buf_v, acc_sh.at[pl.ds(s * RPT, RPT)])
        plsc.subcore_barrier()
        base = wid * epw

        def body(i, carry):
            off = pl.multiple_of(base + i * G, G)
            pltpu.sync_copy(dst_hbm.at[pl.ds(off, G)], idx_v)
            pltpu.sync_copy(upd_v, acc_sh.at[idx_v], add=True)
            return carry

        lax.fori_loop(0, ch, body, 0)
        plsc.subcore_barrier()
        pltpu.sync_copy(acc_sh.at[pl.ds(s * RPT, RPT)], buf_v)
        pltpu.sync_copy(buf_v, out_hbm.at[pl.ds(c * NP + s * RPT, RPT)])

    return k


@functools.lru_cache(maxsize=None)
def _sc_t(e_pad):
    epw = e_pad // NW
    ch = epw // G

    @functools.partial(
        pl.kernel,
        out_type=jax.ShapeDtypeStruct((NC * NP,), jnp.float32),
        mesh=_mesh(),
        compiler_params=pltpu.CompilerParams(needs_layout_passes=False),
        scratch_types=[
            pltpu.VMEM((G,), jnp.int32),
            pltpu.VMEM((G,), jnp.int32),
            pltpu.VMEM((G,), jnp.float32),
            pltpu.VMEM((NP,), jnp.float32),
            pltpu.VMEM((RPT,), jnp.float32),
            pltpu.VMEM_SHARED((NP,), jnp.float32),
        ],
    )
    def k(src_hbm, dst_hbm, dis_hbm, out_hbm,
          sidx_v, didx_v, upd_v, dis_v, buf_v, acc_sh):
        c = lax.axis_index("c")
        s = lax.axis_index("s")
        wid = s * NC + c
        pltpu.sync_copy(dis_hbm, dis_v)
        zeros16 = jnp.zeros((16,), jnp.float32)

        def fill_buf(i, carry):
            buf_v[pl.ds(16 * i, 16)] = zeros16
            return carry

        lax.fori_loop(0, RPT // 16, fill_buf, 0)
        pltpu.sync_copy(buf_v, acc_sh.at[pl.ds(s * RPT, RPT)])
        plsc.subcore_barrier()
        base = wid * epw

        def body(i, carry):
            off = pl.multiple_of(base + i * G, G)
            pltpu.sync_copy(dst_hbm.at[pl.ds(off, G)], didx_v)
            pltpu.sync_copy(src_hbm.at[pl.ds(off, G)], sidx_v)
            for j in range(G // 16):
                dvec = plsc.load_gather(dis_v, [didx_v[pl.ds(16 * j, 16)]])
                upd_v[pl.ds(16 * j, 16)] = dvec
            pltpu.sync_copy(upd_v, acc_sh.at[sidx_v], add=True)
            return carry

        lax.fori_loop(0, ch, body, 0)
        plsc.subcore_barrier()
        pltpu.sync_copy(acc_sh.at[pl.ds(s * RPT, RPT)], buf_v)
        pltpu.sync_copy(buf_v, out_hbm.at[pl.ds(c * NP + s * RPT, RPT)])

    return k


@functools.lru_cache(maxsize=None)
def _sc_mp_edges(e_pad):
    epw = e_pad // NW
    ch = epw // G

    @functools.partial(
        pl.kernel,
        out_type=jax.ShapeDtypeStruct((NC * NP, 128), jnp.float32),
        mesh=_mesh(),
        scratch_types=[
            pltpu.VMEM((G,), jnp.int32),
            pltpu.VMEM((G,), jnp.int32),
            pltpu.VMEM((G, 128), jnp.float32),
            pltpu.VMEM_SHARED((NP, 128), jnp.float32),
            pltpu.SemaphoreType.DMA,
        ],
    )
    def k(table_hbm, src_hbm, dst_hbm, zeros_hbm, out_hbm,
          sidx_v, didx_v, rows_v, acc_sh, sem):
        c = lax.axis_index("c")
        s = lax.axis_index("s")
        wid = s * NC + c
        pltpu.sync_copy(zeros_hbm, rows_v)
        for r in range(RPT // G):
            pltpu.sync_copy(rows_v, acc_sh.at[pl.ds(s * RPT + r * G, G)])
        plsc.subcore_barrier()
        base = wid * epw

        def body(i, carry):
            off = pl.multiple_of(base + i * G, G)
            pltpu.sync_copy(src_hbm.at[pl.ds(off, G)], sidx_v)
            pltpu.async_copy(table_hbm.at[sidx_v], rows_v, sem).wait()
            pltpu.sync_copy(dst_hbm.at[pl.ds(off, G)], didx_v)
            pltpu.sync_copy(rows_v, acc_sh.at[didx_v], add=True)
            return carry

        lax.fori_loop(0, ch, body, 0)
        plsc.subcore_barrier()
        for r in range(RPT // G):
            pltpu.sync_copy(acc_sh.at[pl.ds(s * RPT + r * G, G)], rows_v)
            pltpu.sync_copy(rows_v,
                            out_hbm.at[pl.ds(c * NP + s * RPT + r * G, G)])

    return k


@functools.lru_cache(maxsize=None)
def _sc_mp_feats(e_pad):
    ept = e_pad // NS
    ch = ept // G

    @functools.partial(
        pl.kernel,
        out_type=jax.ShapeDtypeStruct((NC * NP, 128), jnp.float32),
        mesh=_mesh(),
        scratch_types=[
            pltpu.VMEM((G,), jnp.int32),
            pltpu.VMEM((G,), jnp.int32),
            pltpu.VMEM((G, 128), jnp.float32),
            pltpu.VMEM_SHARED((NP, 128), jnp.float32),
            pltpu.SemaphoreType.DMA,
        ],
    )
    def k(ta_hbm, tb_hbm, src_hbm, dst_hbm, zeros_hbm, out_hbm,
          sidx_v, didx_v, rows_v, acc_sh, sem):
        c = lax.axis_index("c")
        s = lax.axis_index("s")
        pltpu.sync_copy(zeros_hbm, rows_v)
        for r in range(RPT // G):
            pltpu.sync_copy(rows_v, acc_sh.at[pl.ds(s * RPT + r * G, G)])
        plsc.subcore_barrier()
        base = s * ept

        def run(tbl):
            def body(i, carry):
                off = pl.multiple_of(base + i * G, G)
                pltpu.sync_copy(src_hbm.at[pl.ds(off, G)], sidx_v)
                pltpu.async_copy(tbl.at[sidx_v], rows_v, sem).wait()
                pltpu.sync_copy(dst_hbm.at[pl.ds(off, G)], didx_v)
                pltpu.sync_copy(rows_v, acc_sh.at[didx_v], add=True)
                return carry

            lax.fori_loop(0, ch, body, 0)

        @pl.when(c == 0)
        def _():
            run(ta_hbm)

        @pl.when(c == 1)
        def _():
            run(tb_hbm)

        plsc.subcore_barrier()
        for r in range(RPT // G):
            pltpu.sync_copy(acc_sh.at[pl.ds(s * RPT + r * G, G)], rows_v)
            pltpu.sync_copy(rows_v,
                            out_hbm.at[pl.ds(c * NP + s * RPT + r * G, G)])

    return k


def _t1a_body(degp_ref, dis_ref):
    deg = degp_ref[0:1, :] + degp_ref[1:2, :] + 1.0
    dis = lax.rsqrt(deg)
    cols = lax.broadcasted_iota(jnp.int32, (1, NP), 1)
    dis_ref[...] = jnp.where(cols < N, dis, 0.0)


def _t1a(degp):
    return pl.pallas_call(
        _t1a_body,
        out_shape=jax.ShapeDtypeStruct((1, NP), jnp.float32),
    )(degp)


def _t1b_body(dis_ref, x_ref, xt_ref):
    xt_ref[...] = dis_ref[...] * x_ref[...]


def _t1b(dis_col, x):
    return pl.pallas_call(
        _t1b_body,
        grid=(GRID,),
        in_specs=[
            pl.BlockSpec((R, 1), lambda i: (i, 0)),
            pl.BlockSpec((R, 128), lambda i: (i, 0)),
        ],
        out_specs=pl.BlockSpec((R, 128), lambda i: (i, 0)),
        out_shape=jax.ShapeDtypeStruct((N, 128), jnp.float32),
    )(dis_col, x)


def _mm_body(sp_ref, h_ref, dis_ref, w_ref, b_ref, z_ref, st_ref, acc):
    i = pl.program_id(0)
    d = dis_ref[...]
    a = d * (sp_ref[0] + sp_ref[1] + h_ref[...])
    z = jnp.dot(a, w_ref[...], preferred_element_type=jnp.float32) + b_ref[...]
    z_ref[...] = z

    @pl.when(i == 0)
    def _():
        acc[...] = jnp.zeros_like(acc)

    acc[0:1, :] += jnp.sum(z, axis=0, keepdims=True)
    acc[1:2, :] += jnp.sum(z * z, axis=0, keepdims=True)
    st_ref[...] = acc[...]


def _t2(s1p, xt, dis_col, w1, b1):
    return pl.pallas_call(
        _mm_body,
        grid=(GRID,),
        in_specs=[
            pl.BlockSpec((NC, R, 128), lambda i: (0, i, 0)),
            pl.BlockSpec((R, 128), lambda i: (i, 0)),
            pl.BlockSpec((R, 1), lambda i: (i, 0)),
            pl.BlockSpec((128, 256), lambda i: (0, 0)),
            pl.BlockSpec((1, 256), lambda i: (0, 0)),
        ],
        out_specs=[
            pl.BlockSpec((R, 256), lambda i: (i, 0)),
            pl.BlockSpec((2, 256), lambda i: (0, 0)),
        ],
        out_shape=[jax.ShapeDtypeStruct((N, 256), jnp.float32),
                   jax.ShapeDtypeStruct((2, 256), jnp.float32)],
        scratch_shapes=[pltpu.VMEM((2, 256), jnp.float32)],
    )(s1p, xt, dis_col, w1, b1)


def _t3_body(z_ref, st_ref, g_ref, be_ref, dis_ref, ha_ref, hb_ref):
    mu = st_ref[0:1, :] * (1.0 / N)
    var = st_ref[1:2, :] * (1.0 / N) - mu * mu
    hn = (z_ref[...] - mu) * lax.rsqrt(var + EPS) * g_ref[...] + be_ref[...]
    h = jnp.maximum(hn, 0.0) * dis_ref[...]
    ha_ref[...] = h[:, :128]
    hb_ref[...] = h[:, 128:]


def _t3(z1, st1, g1, be1, dis_col):
    return pl.pallas_call(
        _t3_body,
        grid=(GRID,),
        in_specs=[
            pl.BlockSpec((R, 256), lambda i: (i, 0)),
            pl.BlockSpec((2, 256), lambda i: (0, 0)),
            pl.BlockSpec((1, 256), lambda i: (0, 0)),
            pl.BlockSpec((1, 256), lambda i: (0, 0)),
            pl.BlockSpec((R, 1), lambda i: (i, 0)),
        ],
        out_specs=[
            pl.BlockSpec((R, 128), lambda i: (i, 0)),
            pl.BlockSpec((R, 128), lambda i: (i, 0)),
        ],
        out_shape=[jax.ShapeDtypeStruct((N, 128), jnp.float32),
                   jax.ShapeDtypeStruct((N, 128), jnp.float32)],
    )(z1, st1, g1, be1, dis_col)


def _t4_body(sp_ref, ha_ref, hb_ref, dis_ref, w_ref, b_ref, z_ref, st_ref,
             acc):
    i = pl.program_id(0)
    d = dis_ref[...]
    a = jnp.concatenate(
        [d * (sp_ref[0] + ha_ref[...]), d * (sp_ref[1] + hb_ref[...])], axis=1)
    z = jnp.dot(a, w_ref[...], preferred_element_type=jnp.float32) + b_ref[...]
    z_ref[...] = z

    @pl.when(i == 0)
    def _():
        acc[...] = jnp.zeros_like(acc)

    acc[0:1, :] += jnp.sum(z, axis=0, keepdims=True)
    acc[1:2, :] += jnp.sum(z * z, axis=0, keepdims=True)
    st_ref[...] = acc[...]


def _t4(s2p, ha, hb, dis_col, w2, b2):
    return pl.pallas_call(
        _t4_body,
        grid=(GRID,),
        in_specs=[
            pl.BlockSpec((NC, R, 128), lambda i: (0, i, 0)),
            pl.BlockSpec((R, 128), lambda i: (i, 0)),
            pl.BlockSpec((R, 128), lambda i: (i, 0)),
            pl.BlockSpec((R, 1), lambda i: (i, 0)),
            pl.BlockSpec((256, 256), lambda i: (0, 0)),
            pl.BlockSpec((1, 256), lambda i: (0, 0)),
        ],
        out_specs=[
            pl.BlockSpec((R, 256), lambda i: (i, 0)),
            pl.BlockSpec((2, 256), lambda i: (0, 0)),
        ],
        out_shape=[jax.ShapeDtypeStruct((N, 256), jnp.float32),
                   jax.ShapeDtypeStruct((2, 256), jnp.float32)],
        scratch_shapes=[pltpu.VMEM((2, 256), jnp.float32)],
    )(s2p, ha, hb, dis_col, w2, b2)


def _t5_body(z_ref, st_ref, g_ref, be_ref, dis_ref, tp_ref, w3_ref, b3_ref,
             out_ref, acc):
    i = pl.program_id(0)
    mu = st_ref[0:1, :] * (1.0 / N)
    var = st_ref[1:2, :] * (1.0 / N) - mu * mu
    h = jnp.maximum(
        (z_ref[...] - mu) * lax.rsqrt(var + EPS) * g_ref[...] + be_ref[...],
        0.0)
    d = dis_ref[...]
    t = tp_ref[0] + tp_ref[1]
    cv = d * (t + d)

    @pl.when(i == 0)
    def _():
        acc[...] = jnp.zeros_like(acc)

    acc[...] += jnp.sum(cv * h, axis=0, keepdims=True)
    out_ref[...] = jnp.dot(acc[...] * (1.0 / N), w3_ref[...],
                           preferred_element_type=jnp.float32) + b3_ref[...]


def _t5(z2, st2, g2, be2, dis_col, tp, w3, b3):
    return pl.pallas_call(
        _t5_body,
        grid=(GRID,),
        in_specs=[
            pl.BlockSpec((R, 256), lambda i: (i, 0)),
            pl.BlockSpec((2, 256), lambda i: (0, 0)),
            pl.BlockSpec((1, 256), lambda i: (0, 0)),
            pl.BlockSpec((1, 256), lambda i: (0, 0)),
            pl.BlockSpec((R, 1), lambda i: (i, 0)),
            pl.BlockSpec((NC, R, 1), lambda i: (0, i, 0)),
            pl.BlockSpec((256, 128), lambda i: (0, 0)),
            pl.BlockSpec((1, 128), lambda i: (0, 0)),
        ],
        out_specs=pl.BlockSpec((1, 128), lambda i: (0, 0)),
        out_shape=jax.ShapeDtypeStruct((1, 128), jnp.float32),
        scratch_shapes=[pltpu.VMEM((1, 256), jnp.float32)],
    )(z2, st2, g2, be2, dis_col, tp, w3, b3)


def kernel(x, edge_index, W1, b1, g1, be1, W2, b2, g2, be2, W3, b3):
    src = edge_index[0]
    dst = edge_index[1]
    e = src.shape[0]
    e_pad = -(-e // (NW * G)) * (NW * G)
    pidx = jnp.arange(e_pad - e, dtype=jnp.int32) % 16
    src_p = jnp.concatenate([src, pidx])
    dst_p = jnp.concatenate([dst, N + pidx])
    zeros_128 = jnp.zeros((G, 128), jnp.float32)

    degp = jnp.reshape(_sc_deg(e_pad)(dst_p), (NC, NP))
    dis_row = _t1a(degp)
    dis_flat = jnp.reshape(dis_row, (NP,))
    dis_col = jnp.reshape(dis_row, (NP, 1))
    xt = _t1b(dis_col, x)
    tp = jnp.reshape(_sc_t(e_pad)(src_p, dst_p, dis_flat), (NC, NP, 1))
    s1p = jnp.reshape(_sc_mp_edges(e_pad)(xt, src_p, dst_p, zeros_128),
                      (NC, NP, 128))
    z1, st1 = _t2(s1p, xt, dis_col, W1, b1.reshape(1, -1))
    ha, hb = _t3(z1, st1, g1.reshape(1, -1), be1.reshape(1, -1), dis_col)
    s2p = jnp.reshape(_sc_mp_feats(e_pad)(ha, hb, src_p, dst_p, zeros_128),
                      (NC, NP, 128))
    z2, st2 = _t4(s2p, ha, hb, dis_col, W2, b2.reshape(1, -1))
    return _t5(z2, st2, g2.reshape(1, -1), be2.reshape(1, -1), dis_col, tp,
               W3, b3.reshape(1, -1))


# speedup vs baseline: 21.9136x; 1.4849x over previous
import functools

import jax
import jax.numpy as jnp
from jax import lax
from jax.experimental import pallas as pl
from jax.experimental.pallas import tpu as pltpu
from jax.experimental.pallas import tpu_sc as plsc

N = 10000
EPS = 1e-5
NC = 2
NS = 16
NW = NC * NS
G = 128
NP = 10240
RPT = NP // NS
R = 400
GRID = N // R

_mesh = lambda: plsc.VectorSubcoreMesh(
    core_axis_name="c", subcore_axis_name="s", num_cores=NC, num_subcores=NS)


@functools.lru_cache(maxsize=None)
def _sc_deg(e_pad):
    epw = e_pad // NW
    ch = epw // G

    @functools.partial(
        pl.kernel,
        out_type=jax.ShapeDtypeStruct((NC * NP,), jnp.float32),
        mesh=_mesh(),
        scratch_types=[
            pltpu.VMEM((G,), jnp.int32),
            pltpu.VMEM((G,), jnp.float32),
            pltpu.VMEM((RPT,), jnp.float32),
            pltpu.VMEM_SHARED((NP,), jnp.float32),
        ],
    )
    def k(dst_hbm, out_hbm, idx_v, upd_v, buf_v, acc_sh):
        c = lax.axis_index("c")
        s = lax.axis_index("s")
        wid = s * NC + c
        ones16 = jnp.ones((16,), jnp.float32)
        zeros16 = jnp.zeros((16,), jnp.float32)

        def fill_upd(i, carry):
            upd_v[pl.ds(16 * i, 16)] = ones16
            return carry

        lax.fori_loop(0, G // 16, fill_upd, 0)

        def fill_buf(i, carry):
            buf_v[pl.ds(16 * i, 16)] = zeros16
            return carry

        lax.fori_loop(0, RPT // 16, fill_buf, 0)
        pltpu.sync_copy(buf_v, acc_sh.at[pl.ds(s * RPT, RPT)])
        plsc.subcore_barrier()
        base = wid * epw

        def body(i, carry):
            off = pl.multiple_of(base + i * G, G)
            pltpu.sync_copy(dst_hbm.at[pl.ds(off, G)], idx_v)
            pltpu.sync_copy(upd_v, acc_sh.at[idx_v], add=True)
            return carry

        lax.fori_loop(0, ch, body, 0)
        plsc.subcore_barrier()
        pltpu.sync_copy(acc_sh.at[pl.ds(s * RPT, RPT)], buf_v)
        pltpu.sync_copy(buf_v, out_hbm.at[pl.ds(c * NP + s * RPT, RPT)])

    return k


@functools.lru_cache(maxsize=None)
def _sc_mp_edges(e_pad):
    epw = e_pad // NW
    ch = epw // G
    assert ch % 2 == 0

    @functools.partial(
        pl.kernel,
        out_type=[jax.ShapeDtypeStruct((NC * NP, 128), jnp.float32),
                  jax.ShapeDtypeStruct((NC * NP,), jnp.float32)],
        mesh=_mesh(),
        compiler_params=pltpu.CompilerParams(needs_layout_passes=False),
        scratch_types=[
            pltpu.VMEM((G,), jnp.int32),
            pltpu.VMEM((G,), jnp.int32),
            pltpu.VMEM((G,), jnp.int32),
            pltpu.VMEM((G,), jnp.int32),
            pltpu.VMEM((G, 128), jnp.float32),
            pltpu.VMEM((G, 128), jnp.float32),
            pltpu.VMEM((G,), jnp.float32),
            pltpu.VMEM((G,), jnp.float32),
            pltpu.VMEM((NP,), jnp.float32),
            pltpu.VMEM((RPT,), jnp.float32),
            pltpu.VMEM_SHARED((NP, 128), jnp.float32),
            pltpu.VMEM_SHARED((NP,), jnp.float32),
            pltpu.SemaphoreType.DMA,
            pltpu.SemaphoreType.DMA,
            pltpu.SemaphoreType.DMA,
            pltpu.SemaphoreType.DMA,
        ],
    )
    def k(table_hbm, src_hbm, dst_hbm, dis_hbm, zeros_hbm, out_hbm, out2_hbm,
          sidx_a, sidx_b, didx_a, didx_b, rows_a, rows_b, upd_a, upd_b,
          dis_v, buf_v, acc_sh, acc2_sh, gsa, gsb, ssa, ssb):
        c = lax.axis_index("c")
        s = lax.axis_index("s")
        wid = s * NC + c
        pltpu.sync_copy(dis_hbm, dis_v)
        pltpu.sync_copy(zeros_hbm, rows_a)
        for r in range(RPT // G):
            pltpu.sync_copy(rows_a, acc_sh.at[pl.ds(s * RPT + r * G, G)])
        zeros16 = jnp.zeros((16,), jnp.float32)

        def fill_buf(i, carry):
            buf_v[pl.ds(16 * i, 16)] = zeros16
            return carry

        lax.fori_loop(0, RPT // 16, fill_buf, 0)
        pltpu.sync_copy(buf_v, acc2_sh.at[pl.ds(s * RPT, RPT)])
        plsc.subcore_barrier()
        base = wid * epw
        bufs = ((sidx_a, didx_a, rows_a, upd_a, gsa, ssa),
                (sidx_b, didx_b, rows_b, upd_b, gsb, ssb))

        def block(kk, carry):
            i0 = 2 * kk
            @pl.when(kk > 0)
            def _():
                for (sv, dv, rv, uv, gs, ss) in bufs:
                    pltpu.make_async_copy(rv, acc_sh.at[dv], ss).wait()

            for b, (sv, dv, rv, uv, gs, ss) in enumerate(bufs):
                off = pl.multiple_of(base + (i0 + b) * G, G)
                pltpu.sync_copy(src_hbm.at[pl.ds(off, G)], sv)
                pltpu.async_copy(table_hbm.at[sv], rv, gs)
                pltpu.sync_copy(dst_hbm.at[pl.ds(off, G)], dv)
            for b, (sv, dv, rv, uv, gs, ss) in enumerate(bufs):
                for j in range(G // 16):
                    uv[pl.ds(16 * j, 16)] = plsc.load_gather(
                        dis_v, [dv[pl.ds(16 * j, 16)]])
                pltpu.sync_copy(uv, acc2_sh.at[sv], add=True)
                pltpu.make_async_copy(table_hbm.at[sv], rv, gs).wait()
                pltpu.async_copy(rv, acc_sh.at[dv], ss, add=True)
            return carry

        lax.fori_loop(0, ch // 2, block, 0)
        for (sv, dv, rv, uv, gs, ss) in bufs:
            pltpu.make_async_copy(rv, acc_sh.at[dv], ss).wait()
        plsc.subcore_barrier()
        for r in range(RPT // G):
            pltpu.sync_copy(acc_sh.at[pl.ds(s * RPT + r * G, G)], rows_a)
            pltpu.sync_copy(rows_a,
                            out_hbm.at[pl.ds(c * NP + s * RPT + r * G, G)])
        pltpu.sync_copy(acc2_sh.at[pl.ds(s * RPT, RPT)], buf_v)
        pltpu.sync_copy(buf_v, out2_hbm.at[pl.ds(c * NP + s * RPT, RPT)])

    return k


@functools.lru_cache(maxsize=None)
def _sc_mp_feats(e_pad):
    ept = e_pad // NS
    ch = ept // G

    assert ch % 2 == 0

    @functools.partial(
        pl.kernel,
        out_type=jax.ShapeDtypeStruct((NC * NP, 128), jnp.float32),
        mesh=_mesh(),
        scratch_types=[
            pltpu.VMEM((G,), jnp.int32),
            pltpu.VMEM((G,), jnp.int32),
            pltpu.VMEM((G,), jnp.int32),
            pltpu.VMEM((G,), jnp.int32),
            pltpu.VMEM((G, 128), jnp.float32),
            pltpu.VMEM((G, 128), jnp.float32),
            pltpu.VMEM_SHARED((NP, 128), jnp.float32),
            pltpu.SemaphoreType.DMA,
            pltpu.SemaphoreType.DMA,
            pltpu.SemaphoreType.DMA,
            pltpu.SemaphoreType.DMA,
        ],
    )
    def k(ta_hbm, tb_hbm, src_hbm, dst_hbm, zeros_hbm, out_hbm,
          sidx_a, sidx_b, didx_a, didx_b, rows_a, rows_b,
          acc_sh, gsa, gsb, ssa, ssb):
        c = lax.axis_index("c")
        s = lax.axis_index("s")
        pltpu.sync_copy(zeros_hbm, rows_a)
        for r in range(RPT // G):
            pltpu.sync_copy(rows_a, acc_sh.at[pl.ds(s * RPT + r * G, G)])
        plsc.subcore_barrier()
        base = s * ept
        bufs = ((sidx_a, didx_a, rows_a, gsa, ssa),
                (sidx_b, didx_b, rows_b, gsb, ssb))

        def run(tbl):
            def block(kk, carry):
                i0 = 2 * kk

                @pl.when(kk > 0)
                def _():
                    for (sv, dv, rv, gs, ss) in bufs:
                        pltpu.make_async_copy(rv, acc_sh.at[dv], ss).wait()

                for b, (sv, dv, rv, gs, ss) in enumerate(bufs):
                    off = pl.multiple_of(base + (i0 + b) * G, G)
                    pltpu.sync_copy(src_hbm.at[pl.ds(off, G)], sv)
                    pltpu.async_copy(tbl.at[sv], rv, gs)
                    pltpu.sync_copy(dst_hbm.at[pl.ds(off, G)], dv)
                for b, (sv, dv, rv, gs, ss) in enumerate(bufs):
                    pltpu.make_async_copy(tbl.at[sv], rv, gs).wait()
                    pltpu.async_copy(rv, acc_sh.at[dv], ss, add=True)
                return carry

            lax.fori_loop(0, ch // 2, block, 0)
            for (sv, dv, rv, gs, ss) in bufs:
                pltpu.make_async_copy(rv, acc_sh.at[dv], ss).wait()

        @pl.when(c == 0)
        def _():
            run(ta_hbm)

        @pl.when(c == 1)
        def _():
            run(tb_hbm)

        plsc.subcore_barrier()
        for r in range(RPT // G):
            pltpu.sync_copy(acc_sh.at[pl.ds(s * RPT + r * G, G)], rows_a)
            pltpu.sync_copy(rows_a,
                            out_hbm.at[pl.ds(c * NP + s * RPT + r * G, G)])

    return k


def _t1a_body(degp_ref, dis_ref):
    deg = degp_ref[0:1, :] + degp_ref[1:2, :] + 1.0
    dis = lax.rsqrt(deg)
    cols = lax.broadcasted_iota(jnp.int32, (1, NP), 1)
    dis_ref[...] = jnp.where(cols < N, dis, 0.0)


def _t1a(degp):
    return pl.pallas_call(
        _t1a_body,
        out_shape=jax.ShapeDtypeStruct((1, NP), jnp.float32),
    )(degp)


def _t1b_body(dis_ref, x_ref, xt_ref):
    xt_ref[...] = dis_ref[...] * x_ref[...]


def _t1b(dis_col, x):
    return pl.pallas_call(
        _t1b_body,
        grid=(GRID,),
        in_specs=[
            pl.BlockSpec((R, 1), lambda i: (i, 0)),
            pl.BlockSpec((R, 128), lambda i: (i, 0)),
        ],
        out_specs=pl.BlockSpec((R, 128), lambda i: (i, 0)),
        out_shape=jax.ShapeDtypeStruct((N, 128), jnp.float32),
    )(dis_col, x)


def _mm_body(sp_ref, h_ref, dis_ref, w_ref, b_ref, z_ref, st_ref, acc):
    i = pl.program_id(0)
    d = dis_ref[...]
    a = d * (sp_ref[0] + sp_ref[1] + h_ref[...])
    z = jnp.dot(a, w_ref[...], preferred_element_type=jnp.float32) + b_ref[...]
    z_ref[...] = z

    @pl.when(i == 0)
    def _():
        acc[...] = jnp.zeros_like(acc)

    acc[0:1, :] += jnp.sum(z, axis=0, keepdims=True)
    acc[1:2, :] += jnp.sum(z * z, axis=0, keepdims=True)
    st_ref[...] = acc[...]


def _t2(s1p, xt, dis_col, w1, b1):
    return pl.pallas_call(
        _mm_body,
        grid=(GRID,),
        in_specs=[
            pl.BlockSpec((NC, R, 128), lambda i: (0, i, 0)),
            pl.BlockSpec((R, 128), lambda i: (i, 0)),
            pl.BlockSpec((R, 1), lambda i: (i, 0)),
            pl.BlockSpec((128, 256), lambda i: (0, 0)),
            pl.BlockSpec((1, 256), lambda i: (0, 0)),
        ],
        out_specs=[
            pl.BlockSpec((R, 256), lambda i: (i, 0)),
            pl.BlockSpec((2, 256), lambda i: (0, 0)),
        ],
        out_shape=[jax.ShapeDtypeStruct((N, 256), jnp.float32),
                   jax.ShapeDtypeStruct((2, 256), jnp.float32)],
        scratch_shapes=[pltpu.VMEM((2, 256), jnp.float32)],
    )(s1p, xt, dis_col, w1, b1)


def _t3_body(z_ref, st_ref, g_ref, be_ref, dis_ref, ha_ref, hb_ref):
    mu = st_ref[0:1, :] * (1.0 / N)
    var = st_ref[1:2, :] * (1.0 / N) - mu * mu
    hn = (z_ref[...] - mu) * lax.rsqrt(var + EPS) * g_ref[...] + be_ref[...]
    h = jnp.maximum(hn, 0.0) * dis_ref[...]
    ha_ref[...] = h[:, :128]
    hb_ref[...] = h[:, 128:]


def _t3(z1, st1, g1, be1, dis_col):
    return pl.pallas_call(
        _t3_body,
        grid=(GRID,),
        in_specs=[
            pl.BlockSpec((R, 256), lambda i: (i, 0)),
            pl.BlockSpec((2, 256), lambda i: (0, 0)),
            pl.BlockSpec((1, 256), lambda i: (0, 0)),
            pl.BlockSpec((1, 256), lambda i: (0, 0)),
            pl.BlockSpec((R, 1), lambda i: (i, 0)),
        ],
        out_specs=[
            pl.BlockSpec((R, 128), lambda i: (i, 0)),
            pl.BlockSpec((R, 128), lambda i: (i, 0)),
        ],
        out_shape=[jax.ShapeDtypeStruct((N, 128), jnp.float32),
                   jax.ShapeDtypeStruct((N, 128), jnp.float32)],
    )(z1, st1, g1, be1, dis_col)


def _t4_body(sp_ref, ha_ref, hb_ref, dis_ref, w_ref, b_ref, z_ref, st_ref,
             acc):
    i = pl.program_id(0)
    d = dis_ref[...]
    a = jnp.concatenate(
        [d * (sp_ref[0] + ha_ref[...]), d * (sp_ref[1] + hb_ref[...])], axis=1)
    z = jnp.dot(a, w_ref[...], preferred_element_type=jnp.float32) + b_ref[...]
    z_ref[...] = z

    @pl.when(i == 0)
    def _():
        acc[...] = jnp.zeros_like(acc)

    acc[0:1, :] += jnp.sum(z, axis=0, keepdims=True)
    acc[1:2, :] += jnp.sum(z * z, axis=0, keepdims=True)
    st_ref[...] = acc[...]


def _t4(s2p, ha, hb, dis_col, w2, b2):
    return pl.pallas_call(
        _t4_body,
        grid=(GRID,),
        in_specs=[
            pl.BlockSpec((NC, R, 128), lambda i: (0, i, 0)),
            pl.BlockSpec((R, 128), lambda i: (i, 0)),
            pl.BlockSpec((R, 128), lambda i: (i, 0)),
            pl.BlockSpec((R, 1), lambda i: (i, 0)),
            pl.BlockSpec((256, 256), lambda i: (0, 0)),
            pl.BlockSpec((1, 256), lambda i: (0, 0)),
        ],
        out_specs=[
            pl.BlockSpec((R, 256), lambda i: (i, 0)),
            pl.BlockSpec((2, 256), lambda i: (0, 0)),
        ],
        out_shape=[jax.ShapeDtypeStruct((N, 256), jnp.float32),
                   jax.ShapeDtypeStruct((2, 256), jnp.float32)],
        scratch_shapes=[pltpu.VMEM((2, 256), jnp.float32)],
    )(s2p, ha, hb, dis_col, w2, b2)


def _t5_body(z_ref, st_ref, g_ref, be_ref, dis_ref, tp_ref, w3_ref, b3_ref,
             out_ref, acc):
    i = pl.program_id(0)
    mu = st_ref[0:1, :] * (1.0 / N)
    var = st_ref[1:2, :] * (1.0 / N) - mu * mu
    h = jnp.maximum(
        (z_ref[...] - mu) * lax.rsqrt(var + EPS) * g_ref[...] + be_ref[...],
        0.0)
    d = dis_ref[...]
    t = tp_ref[0] + tp_ref[1]
    cv = d * (t + d)

    @pl.when(i == 0)
    def _():
        acc[...] = jnp.zeros_like(acc)

    acc[...] += jnp.sum(cv * h, axis=0, keepdims=True)
    out_ref[...] = jnp.dot(acc[...] * (1.0 / N), w3_ref[...],
                           preferred_element_type=jnp.float32) + b3_ref[...]


def _t5(z2, st2, g2, be2, dis_col, tp, w3, b3):
    return pl.pallas_call(
        _t5_body,
        grid=(GRID,),
        in_specs=[
            pl.BlockSpec((R, 256), lambda i: (i, 0)),
            pl.BlockSpec((2, 256), lambda i: (0, 0)),
            pl.BlockSpec((1, 256), lambda i: (0, 0)),
            pl.BlockSpec((1, 256), lambda i: (0, 0)),
            pl.BlockSpec((R, 1), lambda i: (i, 0)),
            pl.BlockSpec((NC, R, 1), lambda i: (0, i, 0)),
            pl.BlockSpec((256, 128), lambda i: (0, 0)),
            pl.BlockSpec((1, 128), lambda i: (0, 0)),
        ],
        out_specs=pl.BlockSpec((1, 128), lambda i: (0, 0)),
        out_shape=jax.ShapeDtypeStruct((1, 128), jnp.float32),
        scratch_shapes=[pltpu.VMEM((1, 256), jnp.float32)],
    )(z2, st2, g2, be2, dis_col, tp, w3, b3)


def kernel(x, edge_index, W1, b1, g1, be1, W2, b2, g2, be2, W3, b3):
    src = edge_index[0]
    dst = edge_index[1]
    e = src.shape[0]
    e_pad = -(-e // (NW * G * 2)) * (NW * G * 2)
    pidx = jnp.arange(e_pad - e, dtype=jnp.int32) % 16
    src_p = jnp.concatenate([src, pidx])
    dst_p = jnp.concatenate([dst, N + pidx])
    zeros_128 = jnp.zeros((G, 128), jnp.float32)

    degp = jnp.reshape(_sc_deg(e_pad)(dst_p), (NC, NP))
    dis_row = _t1a(degp)
    dis_flat = jnp.reshape(dis_row, (NP,))
    dis_col = jnp.reshape(dis_row, (NP, 1))
    xt = _t1b(dis_col, x)
    s1_flat, tp_flat = _sc_mp_edges(e_pad)(xt, src_p, dst_p, dis_flat,
                                           zeros_128)
    s1p = jnp.reshape(s1_flat, (NC, NP, 128))
    tp = jnp.reshape(tp_flat, (NC, NP, 1))
    z1, st1 = _t2(s1p, xt, dis_col, W1, b1.reshape(1, -1))
    ha, hb = _t3(z1, st1, g1.reshape(1, -1), be1.reshape(1, -1), dis_col)
    s2p = jnp.reshape(_sc_mp_feats(e_pad)(ha, hb, src_p, dst_p, zeros_128),
                      (NC, NP, 128))
    z2, st2 = _t4(s2p, ha, hb, dis_col, W2, b2.reshape(1, -1))
    return _t5(z2, st2, g2.reshape(1, -1), be2.reshape(1, -1), dis_col, tp,
               W3, b3.reshape(1, -1))
